# baseline (speedup 1.0000x reference)
import jax
import jax.numpy as jnp
from jax.experimental import pallas as pl


def _mm_body(x_ref, w_ref, o_ref):
    o_ref[...] = jnp.dot(x_ref[...], w_ref[...], preferred_element_type=jnp.float32)


def _matmul(x, w):
    return pl.pallas_call(
        _mm_body,
        out_shape=jax.ShapeDtypeStruct((x.shape[0], w.shape[1]), jnp.float32),
    )(x, w)


def kernel(x, edge_index, W, theta, lin_W, lin_b):
    N = x.shape[0]
    heads, order = theta.shape
    hidden = W.shape[1] // heads
    src = edge_index[0].astype(jnp.int32)
    dst = edge_index[1].astype(jnp.int32)
    ones = jnp.ones((src.shape[0],), dtype=jnp.float32)
    deg_src = jnp.zeros((N,), jnp.float32).at[src].add(ones)
    deg_dst = jnp.zeros((N,), jnp.float32).at[dst].add(ones)
    inv_sqrt_src = 1.0 / jnp.sqrt(jnp.maximum(deg_src, 1.0))
    inv_sqrt_dst = 1.0 / jnp.sqrt(jnp.maximum(deg_dst, 1.0))
    norm = inv_sqrt_src[src] * inv_sqrt_dst[dst]

    h = _matmul(x, W).reshape(N, heads, hidden)

    def prop(v):
        msgs = v[src] * norm[:, None, None]
        return jnp.zeros_like(v).at[dst].add(msgs)

    Tk_prev = h
    Tk = -prop(h)
    out = theta[:, 0][None, :, None] * Tk_prev + theta[:, 1][None, :, None] * Tk
    for k in range(2, order):
        Tk_next = -2.0 * prop(Tk) - Tk_prev
        out = out + theta[:, k][None, :, None] * Tk_next
        Tk_prev, Tk = Tk, Tk_next

    y = jax.nn.elu(out).reshape(N, heads * hidden)
    logits = jax.nn.elu(_matmul(y, lin_W) + lin_b)
    log_probs = jax.nn.log_softmax(logits, axis=1)
    return (log_probs, theta)


# speedup vs baseline: 50.4801x; 50.4801x over previous
import functools

import jax
import jax.numpy as jnp
from jax import lax
from jax.experimental import pallas as pl
from jax.experimental.pallas import tpu as pltpu
from jax.experimental.pallas import tpu_sc as plsc

N = 10000
NP = 10240
E = 320000
F = 128
NC = 2
NS = 16
B = 128
NB = 160
CB = 16
NSB = NB // CB
EPT = NB * B
EPAD = EPT * NS
RPT = NP // NS

_mesh = plsc.VectorSubcoreMesh(core_axis_name="c", subcore_axis_name="s")


@functools.partial(
    pl.kernel,
    out_type=jax.ShapeDtypeStruct((NC * NS, RPT), jnp.float32),
    mesh=_mesh,
    scratch_types=[
        pltpu.VMEM((EPT,), jnp.int32),
        pltpu.VMEM((NP,), jnp.float32),
        pltpu.VMEM((RPT,), jnp.float32),
        pltpu.VMEM((RPT,), jnp.float32),
        pltpu.VMEM_SHARED((NS, NP), jnp.float32),
    ],
    compiler_params=pltpu.CompilerParams(needs_layout_passes=False),
)
def _deg_kernel(src_hbm, dst_hbm, deg_out, idx_v, loc_v, tmp_v, acc_v, sh_all):
    c = lax.axis_index("c")
    s = lax.axis_index("s")

    @pl.when(c == 0)
    def _():
        pltpu.sync_copy(src_hbm.at[s], idx_v)

    @pl.when(c == 1)
    def _():
        pltpu.sync_copy(dst_hbm.at[s], idx_v)

    zeros = jnp.zeros((16,), jnp.float32)

    def zero_body(i, carry):
        loc_v[pl.ds(i * 16, 16)] = zeros
        return carry

    lax.fori_loop(0, NP // 16, zero_body, 0)

    ones = jnp.ones((16,), jnp.float32)

    def scat_body(j, carry):
        idx = idx_v[pl.ds(j * 16, 16)]
        plsc.addupdate_scatter(loc_v, [idx], ones)
        return carry

    lax.fori_loop(0, EPT // 16, scat_body, 0)

    pltpu.sync_copy(loc_v, sh_all.at[s])
    plsc.subcore_barrier()

    def zero_acc(i, carry):
        acc_v[pl.ds(i * 16, 16)] = zeros
        return carry

    lax.fori_loop(0, RPT // 16, zero_acc, 0)

    for t in range(NS):
        pltpu.sync_copy(sh_all.at[t, pl.ds(s * RPT, RPT)], tmp_v)

        def add_body(i, carry):
            sl = pl.ds(i * 16, 16)
            acc_v[sl] = acc_v[sl] + tmp_v[sl]
            return carry

        lax.fori_loop(0, RPT // 16, add_body, 0)

    pltpu.sync_copy(acc_v, deg_out.at[c * NS + s])


@functools.partial(
    pl.kernel,
    out_type=jax.ShapeDtypeStruct((NC * NS, RPT, F), jnp.float32),
    mesh=_mesh,
    scratch_types=[
        pltpu.VMEM((CB, B), jnp.int32),
        pltpu.VMEM((CB, B), jnp.int32),
        pltpu.VMEM((B, F), jnp.float32),
        pltpu.VMEM_SHARED((NP, F), jnp.float32),
        pltpu.SemaphoreType.DMA,
    ],
    compiler_params=pltpu.CompilerParams(needs_layout_passes=False),
)
def _agg_kernel(s0_hbm, s1_hbm, src_hbm, dst_hbm, g_out,
                src_v, dst_v, rows_v, g_sh, gsem):
    c = lax.axis_index("c")
    s = lax.axis_index("s")

    zeros = jnp.zeros((16,), jnp.float32)

    def zrow(i, carry):
        for f in range(F // 16):
            rows_v[i, pl.ds(f * 16, 16)] = zeros
        return carry

    lax.fori_loop(0, B, zrow, 0)

    def zcopy(i, carry):
        pltpu.sync_copy(rows_v, g_sh.at[pl.ds(s * RPT + i * B, B)])
        return carry

    lax.fori_loop(0, RPT // B, zcopy, 0)
    plsc.subcore_barrier()

    def run(table):
        def chunk(sb, carry):
            pltpu.sync_copy(src_hbm.at[s, pl.ds(sb * CB, CB)], src_v)
            pltpu.sync_copy(dst_hbm.at[s, pl.ds(sb * CB, CB)], dst_v)

            def body(j, carry2):
                pltpu.async_copy(table.at[src_v.at[j]], rows_v, gsem).wait()
                pltpu.sync_copy(rows_v, g_sh.at[dst_v.at[j]], add=True)
                return carry2

            lax.fori_loop(0, CB, body, 0)
            return carry

        lax.fori_loop(0, NSB, chunk, 0)

    @pl.when(c == 0)
    def _():
        run(s0_hbm)

    @pl.when(c == 1)
    def _():
        run(s1_hbm)

    plsc.subcore_barrier()
    pltpu.sync_copy(g_sh.at[pl.ds(s * RPT, RPT)], g_out.at[c * NS + s])


def _elu(v):
    return jnp.where(v > 0, v, jnp.exp(jnp.minimum(v, 0.0)) - 1.0)


def _prep_body(x_ref, w_ref, tv_ref, ds_ref, dd_ref,
               s0a_ref, s0b_ref, acc_ref, c_ref, ai_ref):
    degs = jnp.maximum(ds_ref[...], 1.0)
    degd = jnp.maximum(dd_ref[...], 1.0)
    a = lax.rsqrt(degs)
    b = lax.rsqrt(degd)
    c_ref[...] = a * b
    ai_ref[...] = jnp.sqrt(degs)
    h = jnp.dot(x_ref[...], w_ref[...], preferred_element_type=jnp.float32)
    s0 = a[:, None] * h
    s0a_ref[...] = s0[:, :F]
    s0b_ref[...] = s0[:, F:]
    tv = tv_ref[0]
    acc_ref[0] = tv[None, :F] * s0[:, :F]
    acc_ref[1] = tv[None, F:] * s0[:, F:]


_PREP_BLK = 1024


def _prep(x_pad, W, TV, deg_s, deg_d):
    g = NP // _PREP_BLK
    return pl.pallas_call(
        _prep_body,
        grid=(g,),
        in_specs=[
            pl.BlockSpec((_PREP_BLK, 128), lambda i: (i, 0)),
            pl.BlockSpec((128, 2 * F), lambda i: (0, 0)),
            pl.BlockSpec((8, 2 * F), lambda i: (0, 0)),
            pl.BlockSpec((_PREP_BLK,), lambda i: (i,)),
            pl.BlockSpec((_PREP_BLK,), lambda i: (i,)),
        ],
        out_specs=[
            pl.BlockSpec((_PREP_BLK, F), lambda i: (i, 0)),
            pl.BlockSpec((_PREP_BLK, F), lambda i: (i, 0)),
            pl.BlockSpec((2, _PREP_BLK, F), lambda i: (0, i, 0)),
            pl.BlockSpec((_PREP_BLK,), lambda i: (i,)),
            pl.BlockSpec((_PREP_BLK,), lambda i: (i,)),
        ],
        out_shape=[
            jax.ShapeDtypeStruct((NP, F), jnp.float32),
            jax.ShapeDtypeStruct((NP, F), jnp.float32),
            jax.ShapeDtypeStruct((2, NP, F), jnp.float32),
            jax.ShapeDtypeStruct((NP,), jnp.float32),
            jax.ShapeDtypeStruct((NP,), jnp.float32),
        ],
    )(x_pad, W, TV, deg_s, deg_d)


def _rec_body(k, m1, m2, g_ref, sp0_ref, sp1_ref, acc_ref, c_ref, tv_ref,
              sn0_ref, sn1_ref, acco_ref):
    cb = c_ref[...][:, None]
    sn0 = m1 * (cb * g_ref[0]) + m2 * sp0_ref[...]
    sn1 = m1 * (cb * g_ref[1]) + m2 * sp1_ref[...]
    sn0_ref[...] = sn0
    sn1_ref[...] = sn1
    tvk = tv_ref[k]
    acco_ref[0] = acc_ref[0] + tvk[None, :F] * sn0
    acco_ref[1] = acc_ref[1] + tvk[None, F:] * sn1


def _rec(k, m1, m2, g2, sp0, sp1, acc, cvec, TV):
    g = NP // _PREP_BLK
    return pl.pallas_call(
        functools.partial(_rec_body, k, m1, m2),
        grid=(g,),
        in_specs=[
            pl.BlockSpec((2, _PREP_BLK, F), lambda i: (0, i, 0)),
            pl.BlockSpec((_PREP_BLK, F), lambda i: (i, 0)),
            pl.BlockSpec((_PREP_BLK, F), lambda i: (i, 0)),
            pl.BlockSpec((2, _PREP_BLK, F), lambda i: (0, i, 0)),
            pl.BlockSpec((_PREP_BLK,), lambda i: (i,)),
            pl.BlockSpec((8, 2 * F), lambda i: (0, 0)),
        ],
        out_specs=[
            pl.BlockSpec((_PREP_BLK, F), lambda i: (i, 0)),
            pl.BlockSpec((_PREP_BLK, F), lambda i: (i, 0)),
            pl.BlockSpec((2, _PREP_BLK, F), lambda i: (0, i, 0)),
        ],
        out_shape=[
            jax.ShapeDtypeStruct((NP, F), jnp.float32),
            jax.ShapeDtypeStruct((NP, F), jnp.float32),
            jax.ShapeDtypeStruct((2, NP, F), jnp.float32),
        ],
    )(g2, sp0, sp1, acc, cvec, TV)


def _final_body(acc0_ref, acc1_ref, ai_ref, lw_ref, lb_ref, out_ref):
    ai = ai_ref[...][:, None]
    y = jnp.concatenate([ai * acc0_ref[...], ai * acc1_ref[...]], axis=1)
    y = _elu(y)
    logits = jnp.dot(y, lw_ref[...], preferred_element_type=jnp.float32)
    logits = logits + lb_ref[...][None, :]
    el = _elu(logits)
    m = jnp.max(el, axis=1, keepdims=True)
    lse = m + jnp.log(jnp.sum(jnp.exp(el - m), axis=1, keepdims=True))
    out_ref[...] = el - lse


_FIN_BLK = 1024


def _final(acc0, acc1, ainv, lin_W, lin_b):
    g = NP // _FIN_BLK
    return pl.pallas_call(
        _final_body,
        grid=(g,),
        in_specs=[
            pl.BlockSpec((_FIN_BLK, F), lambda i: (i, 0)),
            pl.BlockSpec((_FIN_BLK, F), lambda i: (i, 0)),
            pl.BlockSpec((_FIN_BLK,), lambda i: (i,)),
            pl.BlockSpec((2 * F, 40), lambda i: (0, 0)),
            pl.BlockSpec((40,), lambda i: (0,)),
        ],
        out_specs=pl.BlockSpec((_FIN_BLK, 40), lambda i: (i, 0)),
        out_shape=jax.ShapeDtypeStruct((NP, 40), jnp.float32),
    )(acc0, acc1, ainv, lin_W, lin_b)


def kernel(x, edge_index, W, theta, lin_W, lin_b):
    src = edge_index[0].astype(jnp.int32)
    dst = edge_index[1].astype(jnp.int32)
    pad = EPAD - E
    padv = jnp.full((pad,), N, jnp.int32)
    src_p = jnp.concatenate([src, padv])
    dst_p = jnp.concatenate([dst, padv])
    src3 = src_p.reshape(NS, NB, B)
    dst3 = dst_p.reshape(NS, NB, B)
    srcf = src_p.reshape(NS, EPT)
    dstf = dst_p.reshape(NS, EPT)

    degs = _deg_kernel(srcf, dstf).reshape(NC, NP)

    x_pad = jnp.pad(x, ((0, NP - N), (0, 0)))
    TV = jnp.repeat(theta.T, 32, axis=1)

    s0a, s0b, acc, cvec, ainv = _prep(x_pad, W, TV, degs[0], degs[1])

    sp0, sp1 = s0a, s0b
    sc0, sc1 = s0a, s0b
    order = theta.shape[1]
    for k in range(1, order):
        g2 = _agg_kernel(sc0, sc1, src3, dst3).reshape(NC, NP, F)
        m1 = -1.0 if k == 1 else -2.0
        m2 = 0.0 if k == 1 else -1.0
        sn0, sn1, acc = _rec(k, m1, m2, g2, sp0, sp1, acc, cvec, TV)
        sp0, sp1 = sc0, sc1
        sc0, sc1 = sn0, sn1

    lp = _final(acc[0], acc[1], ainv, lin_W, lin_b)[:N]
    return (lp, theta)


# speedup vs baseline: 57.1906x; 1.1329x over previous
import functools

import jax
import jax.numpy as jnp
from jax import lax
from jax.experimental import pallas as pl
from jax.experimental.pallas import tpu as pltpu
from jax.experimental.pallas import tpu_sc as plsc

N = 10000
NP = 10240
E = 320000
F = 128
NC = 2
NS = 16
B = 128
NB = 160
CB = 16
NSB = NB // CB
EPT = NB * B
EPAD = EPT * NS
RPT = NP // NS

_mesh = plsc.VectorSubcoreMesh(core_axis_name="c", subcore_axis_name="s")


@functools.partial(
    pl.kernel,
    out_type=jax.ShapeDtypeStruct((NC * NS, RPT), jnp.float32),
    mesh=_mesh,
    scratch_types=[
        pltpu.VMEM((EPT,), jnp.int32),
        pltpu.VMEM((NP,), jnp.float32),
        pltpu.VMEM((RPT,), jnp.float32),
        pltpu.VMEM((RPT,), jnp.float32),
        pltpu.VMEM_SHARED((NS, NP), jnp.float32),
    ],
    compiler_params=pltpu.CompilerParams(needs_layout_passes=False),
)
def _deg_kernel(src_hbm, dst_hbm, deg_out, idx_v, loc_v, tmp_v, acc_v, sh_all):
    c = lax.axis_index("c")
    s = lax.axis_index("s")

    @pl.when(c == 0)
    def _():
        pltpu.sync_copy(src_hbm.at[s], idx_v)

    @pl.when(c == 1)
    def _():
        pltpu.sync_copy(dst_hbm.at[s], idx_v)

    zeros = jnp.zeros((16,), jnp.float32)

    def zero_body(i, carry):
        loc_v[pl.ds(i * 16, 16)] = zeros
        return carry

    lax.fori_loop(0, NP // 16, zero_body, 0)

    ones = jnp.ones((16,), jnp.float32)

    def scat_body(j, carry):
        idx = idx_v[pl.ds(j * 16, 16)]
        plsc.addupdate_scatter(loc_v, [idx], ones)
        return carry

    lax.fori_loop(0, EPT // 16, scat_body, 0)

    pltpu.sync_copy(loc_v, sh_all.at[s])
    plsc.subcore_barrier()

    def zero_acc(i, carry):
        acc_v[pl.ds(i * 16, 16)] = zeros
        return carry

    lax.fori_loop(0, RPT // 16, zero_acc, 0)

    for t in range(NS):
        pltpu.sync_copy(sh_all.at[t, pl.ds(s * RPT, RPT)], tmp_v)

        def add_body(i, carry):
            sl = pl.ds(i * 16, 16)
            acc_v[sl] = acc_v[sl] + tmp_v[sl]
            return carry

        lax.fori_loop(0, RPT // 16, add_body, 0)

    pltpu.sync_copy(acc_v, deg_out.at[c * NS + s])


@functools.partial(
    pl.kernel,
    out_type=jax.ShapeDtypeStruct((NC * NS, RPT, F), jnp.float32),
    mesh=_mesh,
    scratch_types=[
        pltpu.VMEM((CB, B), jnp.int32),
        pltpu.VMEM((CB, B), jnp.int32),
        pltpu.VMEM((2, B, F), jnp.float32),
        pltpu.VMEM_SHARED((NP, F), jnp.float32),
        pltpu.SemaphoreType.DMA,
        pltpu.SemaphoreType.DMA,
    ],
    compiler_params=pltpu.CompilerParams(needs_layout_passes=False),
)
def _agg_kernel(s0_hbm, s1_hbm, src_hbm, dst_hbm, g_out,
                src_v, dst_v, rows_v, g_sh, gsem, ssem):
    c = lax.axis_index("c")
    s = lax.axis_index("s")

    zeros = jnp.zeros((16,), jnp.float32)

    def zrow(i, carry):
        for f in range(F // 16):
            rows_v[0, i, pl.ds(f * 16, 16)] = zeros
        return carry

    lax.fori_loop(0, B, zrow, 0)

    def zcopy(i, carry):
        pltpu.sync_copy(rows_v.at[0], g_sh.at[pl.ds(s * RPT + i * B, B)])
        return carry

    lax.fori_loop(0, RPT // B, zcopy, 0)
    plsc.subcore_barrier()

    def run(table):
        def chunk(sb, carry):
            pltpu.sync_copy(src_hbm.at[s, pl.ds(sb * CB, CB)], src_v)
            pltpu.sync_copy(dst_hbm.at[s, pl.ds(sb * CB, CB)], dst_v)
            g = {}
            sc = {}
            g[0] = pltpu.async_copy(table.at[src_v.at[0]], rows_v.at[0], gsem)
            for j in range(CB):
                p = j % 2
                g[j].wait()
                sc[j] = pltpu.async_copy(
                    rows_v.at[p], g_sh.at[dst_v.at[j]], ssem, add=True)
                if j + 1 < CB:
                    if j >= 1:
                        sc[j - 1].wait()
                    g[j + 1] = pltpu.async_copy(
                        table.at[src_v.at[j + 1]], rows_v.at[1 - p], gsem)
            sc[CB - 2].wait()
            sc[CB - 1].wait()
            return carry

        lax.fori_loop(0, NSB, chunk, 0)

    @pl.when(c == 0)
    def _():
        run(s0_hbm)

    @pl.when(c == 1)
    def _():
        run(s1_hbm)

    plsc.subcore_barrier()
    pltpu.sync_copy(g_sh.at[pl.ds(s * RPT, RPT)], g_out.at[c * NS + s])


def _elu(v):
    return jnp.where(v > 0, v, jnp.exp(jnp.minimum(v, 0.0)) - 1.0)


def _prep_body(x_ref, w_ref, tv_ref, ds_ref, dd_ref,
               s0a_ref, s0b_ref, acc_ref, c_ref, ai_ref):
    degs = jnp.maximum(ds_ref[...], 1.0)
    degd = jnp.maximum(dd_ref[...], 1.0)
    a = lax.rsqrt(degs)
    b = lax.rsqrt(degd)
    c_ref[...] = a * b
    ai_ref[...] = jnp.sqrt(degs)
    h = jnp.dot(x_ref[...], w_ref[...], preferred_element_type=jnp.float32)
    s0 = a[:, None] * h
    s0a_ref[...] = s0[:, :F]
    s0b_ref[...] = s0[:, F:]
    tv = tv_ref[0]
    acc_ref[0] = tv[None, :F] * s0[:, :F]
    acc_ref[1] = tv[None, F:] * s0[:, F:]


_PREP_BLK = 1024


def _prep(x_pad, W, TV, deg_s, deg_d):
    g = NP // _PREP_BLK
    return pl.pallas_call(
        _prep_body,
        grid=(g,),
        in_specs=[
            pl.BlockSpec((_PREP_BLK, 128), lambda i: (i, 0)),
            pl.BlockSpec((128, 2 * F), lambda i: (0, 0)),
            pl.BlockSpec((8, 2 * F), lambda i: (0, 0)),
            pl.BlockSpec((_PREP_BLK,), lambda i: (i,)),
            pl.BlockSpec((_PREP_BLK,), lambda i: (i,)),
        ],
        out_specs=[
            pl.BlockSpec((_PREP_BLK, F), lambda i: (i, 0)),
            pl.BlockSpec((_PREP_BLK, F), lambda i: (i, 0)),
            pl.BlockSpec((2, _PREP_BLK, F), lambda i: (0, i, 0)),
            pl.BlockSpec((_PREP_BLK,), lambda i: (i,)),
            pl.BlockSpec((_PREP_BLK,), lambda i: (i,)),
        ],
        out_shape=[
            jax.ShapeDtypeStruct((NP, F), jnp.float32),
            jax.ShapeDtypeStruct((NP, F), jnp.float32),
            jax.ShapeDtypeStruct((2, NP, F), jnp.float32),
            jax.ShapeDtypeStruct((NP,), jnp.float32),
            jax.ShapeDtypeStruct((NP,), jnp.float32),
        ],
    )(x_pad, W, TV, deg_s, deg_d)


def _rec_body(k, m1, m2, g_ref, sp0_ref, sp1_ref, acc_ref, c_ref, tv_ref,
              sn0_ref, sn1_ref, acco_ref):
    cb = c_ref[...][:, None]
    sn0 = m1 * (cb * g_ref[0]) + m2 * sp0_ref[...]
    sn1 = m1 * (cb * g_ref[1]) + m2 * sp1_ref[...]
    sn0_ref[...] = sn0
    sn1_ref[...] = sn1
    tvk = tv_ref[k]
    acco_ref[0] = acc_ref[0] + tvk[None, :F] * sn0
    acco_ref[1] = acc_ref[1] + tvk[None, F:] * sn1


def _rec(k, m1, m2, g2, sp0, sp1, acc, cvec, TV):
    g = NP // _PREP_BLK
    return pl.pallas_call(
        functools.partial(_rec_body, k, m1, m2),
        grid=(g,),
        in_specs=[
            pl.BlockSpec((2, _PREP_BLK, F), lambda i: (0, i, 0)),
            pl.BlockSpec((_PREP_BLK, F), lambda i: (i, 0)),
            pl.BlockSpec((_PREP_BLK, F), lambda i: (i, 0)),
            pl.BlockSpec((2, _PREP_BLK, F), lambda i: (0, i, 0)),
            pl.BlockSpec((_PREP_BLK,), lambda i: (i,)),
            pl.BlockSpec((8, 2 * F), lambda i: (0, 0)),
        ],
        out_specs=[
            pl.BlockSpec((_PREP_BLK, F), lambda i: (i, 0)),
            pl.BlockSpec((_PREP_BLK, F), lambda i: (i, 0)),
            pl.BlockSpec((2, _PREP_BLK, F), lambda i: (0, i, 0)),
        ],
        out_shape=[
            jax.ShapeDtypeStruct((NP, F), jnp.float32),
            jax.ShapeDtypeStruct((NP, F), jnp.float32),
            jax.ShapeDtypeStruct((2, NP, F), jnp.float32),
        ],
    )(g2, sp0, sp1, acc, cvec, TV)


def _final_body(acc0_ref, acc1_ref, ai_ref, lw_ref, lb_ref, out_ref):
    ai = ai_ref[...][:, None]
    y = jnp.concatenate([ai * acc0_ref[...], ai * acc1_ref[...]], axis=1)
    y = _elu(y)
    logits = jnp.dot(y, lw_ref[...], preferred_element_type=jnp.float32)
    logits = logits + lb_ref[...][None, :]
    el = _elu(logits)
    m = jnp.max(el, axis=1, keepdims=True)
    lse = m + jnp.log(jnp.sum(jnp.exp(el - m), axis=1, keepdims=True))
    out_ref[...] = el - lse


_FIN_BLK = 1024


def _final(acc0, acc1, ainv, lin_W, lin_b):
    g = NP // _FIN_BLK
    return pl.pallas_call(
        _final_body,
        grid=(g,),
        in_specs=[
            pl.BlockSpec((_FIN_BLK, F), lambda i: (i, 0)),
            pl.BlockSpec((_FIN_BLK, F), lambda i: (i, 0)),
            pl.BlockSpec((_FIN_BLK,), lambda i: (i,)),
            pl.BlockSpec((2 * F, 40), lambda i: (0, 0)),
            pl.BlockSpec((40,), lambda i: (0,)),
        ],
        out_specs=pl.BlockSpec((_FIN_BLK, 40), lambda i: (i, 0)),
        out_shape=jax.ShapeDtypeStruct((NP, 40), jnp.float32),
    )(acc0, acc1, ainv, lin_W, lin_b)


def kernel(x, edge_index, W, theta, lin_W, lin_b):
    src = edge_index[0].astype(jnp.int32)
    dst = edge_index[1].astype(jnp.int32)
    pad = EPAD - E
    padv = jnp.full((pad,), N, jnp.int32)
    src_p = jnp.concatenate([src, padv])
    dst_p = jnp.concatenate([dst, padv])
    src3 = src_p.reshape(NS, NB, B)
    dst3 = dst_p.reshape(NS, NB, B)
    srcf = src_p.reshape(NS, EPT)
    dstf = dst_p.reshape(NS, EPT)

    degs = _deg_kernel(srcf, dstf).reshape(NC, NP)

    x_pad = jnp.pad(x, ((0, NP - N), (0, 0)))
    TV = jnp.repeat(theta.T, 32, axis=1)

    s0a, s0b, acc, cvec, ainv = _prep(x_pad, W, TV, degs[0], degs[1])

    sp0, sp1 = s0a, s0b
    sc0, sc1 = s0a, s0b
    order = theta.shape[1]
    for k in range(1, order):
        g2 = _agg_kernel(sc0, sc1, src3, dst3).reshape(NC, NP, F)
        m1 = -1.0 if k == 1 else -2.0
        m2 = 0.0 if k == 1 else -1.0
        sn0, sn1, acc = _rec(k, m1, m2, g2, sp0, sp1, acc, cvec, TV)
        sp0, sp1 = sc0, sc1
        sc0, sc1 = sn0, sn1

    lp = _final(acc[0], acc[1], ainv, lin_W, lin_b)[:N]
    return (lp, theta)


# speedup vs baseline: 104.6615x; 1.8300x over previous
import functools

import jax
import jax.numpy as jnp
from jax import lax
from jax.experimental import pallas as pl
from jax.experimental.pallas import tpu as pltpu
from jax.experimental.pallas import tpu_sc as plsc

N = 10000
NP = 10240
E = 320000
F = 128
NC = 2
NS = 16
B = 120
NB = 168
CB = 8
NSB = NB // CB
NBUF = 3
D = 2
EPT = NB * B
EPAD = EPT * NS
RPT = NP // NS

_mesh = plsc.VectorSubcoreMesh(core_axis_name="c", subcore_axis_name="s")


@functools.partial(
    pl.kernel,
    out_type=jax.ShapeDtypeStruct((NC * NS, RPT), jnp.float32),
    mesh=_mesh,
    scratch_types=[
        pltpu.VMEM((EPT,), jnp.int32),
        pltpu.VMEM((NP,), jnp.float32),
        pltpu.VMEM((RPT,), jnp.float32),
        pltpu.VMEM((RPT,), jnp.float32),
        pltpu.VMEM_SHARED((NS, NP), jnp.float32),
    ],
    compiler_params=pltpu.CompilerParams(needs_layout_passes=False),
)
def _deg_kernel(src_hbm, dst_hbm, deg_out, idx_v, loc_v, tmp_v, acc_v, sh_all):
    c = lax.axis_index("c")
    s = lax.axis_index("s")

    @pl.when(c == 0)
    def _():
        pltpu.sync_copy(src_hbm.at[s], idx_v)

    @pl.when(c == 1)
    def _():
        pltpu.sync_copy(dst_hbm.at[s], idx_v)

    zeros = jnp.zeros((16,), jnp.float32)

    def zero_body(i, carry):
        loc_v[pl.ds(i * 16, 16)] = zeros
        return carry

    lax.fori_loop(0, NP // 16, zero_body, 0)

    ones = jnp.ones((16,), jnp.float32)

    def scat_body(j, carry):
        idx = idx_v[pl.ds(j * 16, 16)]
        plsc.addupdate_scatter(loc_v, [idx], ones)
        return carry

    lax.fori_loop(0, EPT // 16, scat_body, 0)

    pltpu.sync_copy(loc_v, sh_all.at[s])
    plsc.subcore_barrier()

    def zero_acc(i, carry):
        acc_v[pl.ds(i * 16, 16)] = zeros
        return carry

    lax.fori_loop(0, RPT // 16, zero_acc, 0)

    for t in range(NS):
        pltpu.sync_copy(sh_all.at[t, pl.ds(s * RPT, RPT)], tmp_v)

        def add_body(i, carry):
            sl = pl.ds(i * 16, 16)
            acc_v[sl] = acc_v[sl] + tmp_v[sl]
            return carry

        lax.fori_loop(0, RPT // 16, add_body, 0)

    pltpu.sync_copy(acc_v, deg_out.at[c * NS + s])


@functools.partial(
    pl.kernel,
    out_type=jax.ShapeDtypeStruct((NC * NS, RPT, F), jnp.float32),
    mesh=_mesh,
    scratch_types=[
        pltpu.VMEM((CB, B), jnp.int32),
        pltpu.VMEM((CB, B), jnp.int32),
        pltpu.VMEM((NBUF, B, F), jnp.float32),
        pltpu.VMEM_SHARED((NP, F), jnp.float32),
        pltpu.SemaphoreType.DMA,
        pltpu.SemaphoreType.DMA,
    ],
    compiler_params=pltpu.CompilerParams(needs_layout_passes=False),
)
def _agg_kernel(s0_hbm, s1_hbm, src_hbm, dst_hbm, g_out,
                src_v, dst_v, rows_v, g_sh, gsem, ssem):
    c = lax.axis_index("c")
    s = lax.axis_index("s")

    zeros = jnp.zeros((16,), jnp.float32)

    def zrow(i, carry):
        for f in range(F // 16):
            rows_v[0, i, pl.ds(f * 16, 16)] = zeros
        return carry

    lax.fori_loop(0, B, zrow, 0)

    def zcopy(i, carry):
        pltpu.sync_copy(rows_v.at[0, pl.ds(0, 80)],
                        g_sh.at[pl.ds(s * RPT + i * 80, 80)])
        return carry

    lax.fori_loop(0, RPT // 80, zcopy, 0)
    plsc.subcore_barrier()

    def run(table):
        def gissue(j):
            return pltpu.async_copy(
                table.at[src_v.at[j]], rows_v.at[j % NBUF], gsem)

        def chunk(sb, carry):
            pltpu.sync_copy(src_hbm.at[s, pl.ds(sb * CB, CB)], src_v)
            pltpu.sync_copy(dst_hbm.at[s, pl.ds(sb * CB, CB)], dst_v)
            g = {}
            sc = {}
            waited = set()
            for j in range(min(D, CB)):
                g[j] = gissue(j)
            for j in range(CB):
                g[j].wait()
                sc[j] = pltpu.async_copy(
                    rows_v.at[j % NBUF], g_sh.at[dst_v.at[j]], ssem, add=True)
                nj = j + D
                if nj < CB:
                    prev = nj - NBUF
                    if prev >= 0:
                        sc[prev].wait()
                        waited.add(prev)
                    g[nj] = gissue(nj)
            for j in range(CB):
                if j not in waited:
                    sc[j].wait()
            return carry

        lax.fori_loop(0, NSB, chunk, 0)

    @pl.when(c == 0)
    def _():
        run(s0_hbm)

    @pl.when(c == 1)
    def _():
        run(s1_hbm)

    plsc.subcore_barrier()
    pltpu.sync_copy(g_sh.at[pl.ds(s * RPT, RPT)], g_out.at[c * NS + s])


def _elu(v):
    return jnp.where(v > 0, v, jnp.exp(jnp.minimum(v, 0.0)) - 1.0)


def _prep_body(x_ref, w_ref, tv_ref, ds_ref, dd_ref,
               s0a_ref, s0b_ref, acc_ref, c_ref, ai_ref):
    degs = jnp.maximum(ds_ref[...], 1.0)
    degd = jnp.maximum(dd_ref[...], 1.0)
    a = lax.rsqrt(degs)
    b = lax.rsqrt(degd)
    c_ref[...] = a * b
    ai_ref[...] = jnp.sqrt(degs)
    h = jnp.dot(x_ref[...], w_ref[...], preferred_element_type=jnp.float32)
    s0 = a[:, None] * h
    s0a_ref[...] = s0[:, :F]
    s0b_ref[...] = s0[:, F:]
    tv = tv_ref[0]
    acc_ref[0] = tv[None, :F] * s0[:, :F]
    acc_ref[1] = tv[None, F:] * s0[:, F:]


_PREP_BLK = 1024


def _prep(x_pad, W, TV, deg_s, deg_d):
    g = NP // _PREP_BLK
    return pl.pallas_call(
        _prep_body,
        grid=(g,),
        in_specs=[
            pl.BlockSpec((_PREP_BLK, 128), lambda i: (i, 0)),
            pl.BlockSpec((128, 2 * F), lambda i: (0, 0)),
            pl.BlockSpec((8, 2 * F), lambda i: (0, 0)),
            pl.BlockSpec((_PREP_BLK,), lambda i: (i,)),
            pl.BlockSpec((_PREP_BLK,), lambda i: (i,)),
        ],
        out_specs=[
            pl.BlockSpec((_PREP_BLK, F), lambda i: (i, 0)),
            pl.BlockSpec((_PREP_BLK, F), lambda i: (i, 0)),
            pl.BlockSpec((2, _PREP_BLK, F), lambda i: (0, i, 0)),
            pl.BlockSpec((_PREP_BLK,), lambda i: (i,)),
            pl.BlockSpec((_PREP_BLK,), lambda i: (i,)),
        ],
        out_shape=[
            jax.ShapeDtypeStruct((NP, F), jnp.float32),
            jax.ShapeDtypeStruct((NP, F), jnp.float32),
            jax.ShapeDtypeStruct((2, NP, F), jnp.float32),
            jax.ShapeDtypeStruct((NP,), jnp.float32),
            jax.ShapeDtypeStruct((NP,), jnp.float32),
        ],
    )(x_pad, W, TV, deg_s, deg_d)


def _rec_body(k, m1, m2, g_ref, sp0_ref, sp1_ref, acc_ref, c_ref, tv_ref,
              sn0_ref, sn1_ref, acco_ref):
    cb = c_ref[...][:, None]
    sn0 = m1 * (cb * g_ref[0]) + m2 * sp0_ref[...]
    sn1 = m1 * (cb * g_ref[1]) + m2 * sp1_ref[...]
    sn0_ref[...] = sn0
    sn1_ref[...] = sn1
    tvk = tv_ref[k]
    acco_ref[0] = acc_ref[0] + tvk[None, :F] * sn0
    acco_ref[1] = acc_ref[1] + tvk[None, F:] * sn1


def _rec(k, m1, m2, g2, sp0, sp1, acc, cvec, TV):
    g = NP // _PREP_BLK
    return pl.pallas_call(
        functools.partial(_rec_body, k, m1, m2),
        grid=(g,),
        in_specs=[
            pl.BlockSpec((2, _PREP_BLK, F), lambda i: (0, i, 0)),
            pl.BlockSpec((_PREP_BLK, F), lambda i: (i, 0)),
            pl.BlockSpec((_PREP_BLK, F), lambda i: (i, 0)),
            pl.BlockSpec((2, _PREP_BLK, F), lambda i: (0, i, 0)),
            pl.BlockSpec((_PREP_BLK,), lambda i: (i,)),
            pl.BlockSpec((8, 2 * F), lambda i: (0, 0)),
        ],
        out_specs=[
            pl.BlockSpec((_PREP_BLK, F), lambda i: (i, 0)),
            pl.BlockSpec((_PREP_BLK, F), lambda i: (i, 0)),
            pl.BlockSpec((2, _PREP_BLK, F), lambda i: (0, i, 0)),
        ],
        out_shape=[
            jax.ShapeDtypeStruct((NP, F), jnp.float32),
            jax.ShapeDtypeStruct((NP, F), jnp.float32),
            jax.ShapeDtypeStruct((2, NP, F), jnp.float32),
        ],
    )(g2, sp0, sp1, acc, cvec, TV)


def _final_body(acc0_ref, acc1_ref, ai_ref, lw_ref, lb_ref, out_ref):
    ai = ai_ref[...][:, None]
    y = jnp.concatenate([ai * acc0_ref[...], ai * acc1_ref[...]], axis=1)
    y = _elu(y)
    logits = jnp.dot(y, lw_ref[...], preferred_element_type=jnp.float32)
    logits = logits + lb_ref[...][None, :]
    el = _elu(logits)
    m = jnp.max(el, axis=1, keepdims=True)
    lse = m + jnp.log(jnp.sum(jnp.exp(el - m), axis=1, keepdims=True))
    out_ref[...] = el - lse


_FIN_BLK = 1024


def _final(acc0, acc1, ainv, lin_W, lin_b):
    g = NP // _FIN_BLK
    return pl.pallas_call(
        _final_body,
        grid=(g,),
        in_specs=[
            pl.BlockSpec((_FIN_BLK, F), lambda i: (i, 0)),
            pl.BlockSpec((_FIN_BLK, F), lambda i: (i, 0)),
            pl.BlockSpec((_FIN_BLK,), lambda i: (i,)),
            pl.BlockSpec((2 * F, 40), lambda i: (0, 0)),
            pl.BlockSpec((40,), lambda i: (0,)),
        ],
        out_specs=pl.BlockSpec((_FIN_BLK, 40), lambda i: (i, 0)),
        out_shape=jax.ShapeDtypeStruct((NP, 40), jnp.float32),
    )(acc0, acc1, ainv, lin_W, lin_b)


def kernel(x, edge_index, W, theta, lin_W, lin_b):
    src = edge_index[0].astype(jnp.int32)
    dst = edge_index[1].astype(jnp.int32)
    pad = EPAD - E
    padv = jnp.full((pad,), N, jnp.int32)
    src_p = jnp.concatenate([src, padv])
    dst_p = jnp.concatenate([dst, padv])
    src3 = src_p.reshape(NS, NB, B)
    dst3 = dst_p.reshape(NS, NB, B)
    srcf = src_p.reshape(NS, EPT)
    dstf = dst_p.reshape(NS, EPT)

    degs = _deg_kernel(srcf, dstf).reshape(NC, NP)

    x_pad = jnp.pad(x, ((0, NP - N), (0, 0)))
    TV = jnp.repeat(theta.T, 32, axis=1)

    s0a, s0b, acc, cvec, ainv = _prep(x_pad, W, TV, degs[0], degs[1])

    sp0, sp1 = s0a, s0b
    sc0, sc1 = s0a, s0b
    order = theta.shape[1]
    for k in range(1, order):
        g2 = _agg_kernel(sc0, sc1, src3, dst3).reshape(NC, NP, F)
        m1 = -1.0 if k == 1 else -2.0
        m2 = 0.0 if k == 1 else -1.0
        sn0, sn1, acc = _rec(k, m1, m2, g2, sp0, sp1, acc, cvec, TV)
        sp0, sp1 = sc0, sc1
        sc0, sc1 = sn0, sn1

    lp = _final(acc[0], acc[1], ainv, lin_W, lin_b)[:N]
    return (lp, theta)
